# baseline (speedup 1.0000x reference)
import jax
import jax.numpy as jnp
from jax.experimental import pallas as pl
from jax.experimental.pallas import tpu as pltpu

EPS = 1e-5
_BR_STATS = 1024
_BR_NORM = 512


def _stats_body(x_ref, s_ref, ss_ref):
    xb = x_ref[...]
    s_row = jnp.sum(xb, axis=0, keepdims=True)
    ss_row = jnp.sum(xb * xb, axis=0, keepdims=True)
    s = jnp.sum(s_row, axis=1, keepdims=True)
    ss = jnp.sum(ss_row, axis=1, keepdims=True)
    s_ref[...] = jnp.broadcast_to(s, (1, 1, 128))
    ss_ref[...] = jnp.broadcast_to(ss, (1, 1, 128))


def _norm_body(scal_ref, x_ref, y_ref):
    mean = scal_ref[0]
    inv_std = scal_ref[1]
    y_ref[...] = (x_ref[...] - mean) * inv_std


def kernel(x, count, mean, m2):
    rows, cols = x.shape
    grid_s = rows // _BR_STATS
    grid_n = rows // _BR_NORM

    s_part, ss_part = pl.pallas_call(
        _stats_body,
        grid=(grid_s,),
        in_specs=[pl.BlockSpec((_BR_STATS, cols), lambda i: (i, 0))],
        out_specs=[
            pl.BlockSpec((1, 1, 128), lambda i: (i, 0, 0)),
            pl.BlockSpec((1, 1, 128), lambda i: (i, 0, 0)),
        ],
        out_shape=[
            jax.ShapeDtypeStruct((grid_s, 1, 128), jnp.float32),
            jax.ShapeDtypeStruct((grid_s, 1, 128), jnp.float32),
        ],
        compiler_params=pltpu.CompilerParams(
            dimension_semantics=("arbitrary",),
            vmem_limit_bytes=56 * 1024 * 1024,
        ),
    )(x)

    n = jnp.float32(rows * cols)
    total_s = jnp.sum(s_part[:, 0, 0])
    total_ss = jnp.sum(ss_part[:, 0, 0])
    bmean = total_s / n
    bm2 = total_ss - total_s * bmean
    new_count = count + n
    delta = bmean - mean
    new_mean = mean + delta * n / new_count
    new_m2 = m2 + bm2 + jnp.square(delta) * count * n / new_count
    var = new_m2 / jnp.maximum(new_count - 1.0, 1.0)
    inv_std = jax.lax.rsqrt(var + EPS)
    scal = jnp.stack([new_mean, inv_std])

    y = pl.pallas_call(
        _norm_body,
        grid=(grid_n,),
        in_specs=[
            pl.BlockSpec(memory_space=pltpu.SMEM),
            pl.BlockSpec((_BR_NORM, cols), lambda i: (i, 0)),
        ],
        out_specs=pl.BlockSpec((_BR_NORM, cols), lambda i: (i, 0)),
        out_shape=jax.ShapeDtypeStruct((rows, cols), jnp.float32),
        compiler_params=pltpu.CompilerParams(
            dimension_semantics=("arbitrary",),
            vmem_limit_bytes=56 * 1024 * 1024,
        ),
    )(scal, x)

    return y, new_count, new_mean, new_m2


# speedup vs baseline: 1.0295x; 1.0295x over previous
import jax
import jax.numpy as jnp
from jax.experimental import pallas as pl
from jax.experimental.pallas import tpu as pltpu

EPS = 1e-5
_BR = 512


def _fused_body(scal_ref, x_ref, y_ref, nc_ref, nm_ref, nm2_ref, acc_ref):
    p = pl.program_id(0)
    g = pl.program_id(1)
    n = jnp.float32(pl.num_programs(1) * _BR * x_ref.shape[1])

    @pl.when(p == 0)
    def _stats():
        xb = x_ref[...]
        s_row = jnp.sum(xb, axis=0, keepdims=True)
        ss_row = jnp.sum(xb * xb, axis=0, keepdims=True)
        s = jnp.sum(s_row, axis=1, keepdims=True)[0, 0]
        ss = jnp.sum(ss_row, axis=1, keepdims=True)[0, 0]
        acc_ref[0] = jnp.where(g == 0, 0.0, acc_ref[0]) + s
        acc_ref[1] = jnp.where(g == 0, 0.0, acc_ref[1]) + ss

    @pl.when(p == 1)
    def _norm():
        count = scal_ref[0]
        mean = scal_ref[1]
        m2 = scal_ref[2]
        total_s = acc_ref[0]
        total_ss = acc_ref[1]
        bmean = total_s / n
        bm2 = total_ss - total_s * bmean
        new_count = count + n
        delta = bmean - mean
        new_mean = mean + delta * n / new_count
        new_m2 = m2 + bm2 + delta * delta * count * n / new_count
        var = new_m2 / jnp.maximum(new_count - 1.0, 1.0)
        inv_std = jax.lax.rsqrt(var + EPS)
        y_ref[...] = (x_ref[...] - new_mean) * inv_std
        nc_ref[0] = new_count
        nm_ref[0] = new_mean
        nm2_ref[0] = new_m2


def kernel(x, count, mean, m2):
    rows, cols = x.shape
    grid = rows // _BR
    scal = jnp.stack([count, mean, m2])

    y, nc, nm, nm2 = pl.pallas_call(
        _fused_body,
        grid=(2, grid),
        in_specs=[
            pl.BlockSpec(memory_space=pltpu.SMEM),
            pl.BlockSpec((_BR, cols), lambda p, g: (g, 0)),
        ],
        out_specs=[
            pl.BlockSpec((_BR, cols), lambda p, g: (p * g, 0)),
            pl.BlockSpec(memory_space=pltpu.SMEM),
            pl.BlockSpec(memory_space=pltpu.SMEM),
            pl.BlockSpec(memory_space=pltpu.SMEM),
        ],
        out_shape=[
            jax.ShapeDtypeStruct((rows, cols), jnp.float32),
            jax.ShapeDtypeStruct((1,), jnp.float32),
            jax.ShapeDtypeStruct((1,), jnp.float32),
            jax.ShapeDtypeStruct((1,), jnp.float32),
        ],
        scratch_shapes=[pltpu.SMEM((2,), jnp.float32)],
        compiler_params=pltpu.CompilerParams(
            dimension_semantics=("arbitrary", "arbitrary"),
            vmem_limit_bytes=56 * 1024 * 1024,
        ),
    )(scal, x)

    return y, nc[0], nm[0], nm2[0]
